# baseline (speedup 1.0000x reference)
import functools

import jax
import jax.numpy as jnp
from jax import lax
from jax.experimental import pallas as pl
from jax.experimental.pallas import tpu as pltpu
from jax.experimental.pallas import tpu_sc as plsc

BATCH = 16384
EMBED_DIM = 64
PAIR = 2 * EMBED_DIM
NC = 2
NS = 16
NW = NC * NS
ROWS_PER_W = BATCH // NW
CHUNK = 128
NCHUNKS = ROWS_PER_W // CHUNK
LANES = 16
DCHUNKS = EMBED_DIM // LANES
GROUPS = CHUNK // LANES


def _sc_similarity(uidx2d, midx2d, user_pairs, movie_pairs):
    mesh = plsc.VectorSubcoreMesh(core_axis_name="c", subcore_axis_name="s")

    @functools.partial(
        pl.kernel,
        mesh=mesh,
        out_type=jax.ShapeDtypeStruct((BATCH,), jnp.float32),
        scratch_types=[
            pltpu.VMEM((NCHUNKS, CHUNK), jnp.int32),
            pltpu.VMEM((NCHUNKS, CHUNK), jnp.int32),
            pltpu.VMEM((NCHUNKS, CHUNK), jnp.int32),
            pltpu.VMEM((NCHUNKS, CHUNK), jnp.int32),
            pltpu.VMEM((2, CHUNK, PAIR), jnp.float32),
            pltpu.VMEM((2, CHUNK, PAIR), jnp.float32),
            pltpu.VMEM((2 * LANES,), jnp.float32),
            pltpu.VMEM((ROWS_PER_W,), jnp.float32),
            pltpu.SemaphoreType.DMA,
            pltpu.SemaphoreType.DMA,
        ],
    )
    def k(uidx_hbm, midx_hbm, utab_hbm, mtab_hbm, out_hbm,
          uidx_v, midx_v, upair_v, mpair_v, ubuf, mbuf, scr, out_v,
          sem0, sem1):
        wid = lax.axis_index("s") * NC + lax.axis_index("c")
        ibase = wid * NCHUNKS

        pltpu.sync_copy(uidx_hbm.at[pl.ds(ibase, NCHUNKS)], uidx_v)
        pltpu.sync_copy(midx_hbm.at[pl.ds(ibase, NCHUNKS)], midx_v)

        def shr_body(j, carry):
            g = j // (CHUNK // LANES)
            o = (j % (CHUNK // LANES)) * LANES
            upair_v[g, pl.ds(o, LANES)] = jnp.right_shift(
                uidx_v[g, pl.ds(o, LANES)], 1)
            mpair_v[g, pl.ds(o, LANES)] = jnp.right_shift(
                midx_v[g, pl.ds(o, LANES)], 1)
            return carry

        lax.fori_loop(0, NCHUNKS * (CHUNK // LANES), shr_body, 0)

        sems = (sem0, sem1)

        def fire(g):
            slot = g % 2
            return (
                pltpu.async_copy(utab_hbm.at[upair_v.at[g]],
                                 ubuf.at[slot], sems[slot]),
                pltpu.async_copy(mtab_hbm.at[mpair_v.at[g]],
                                 mbuf.at[slot], sems[slot]),
            )

        lane_iota = lax.iota(jnp.int32, LANES)

        def compute_chunk(g):
            slot = g % 2
            urows = ubuf.at[slot]
            mrows = mbuf.at[slot]

            def group_body(gg, carry):
                lbase = gg * LANES
                upar = (uidx_v[g, pl.ds(lbase, LANES)] & 1) * EMBED_DIM
                mpar = (midx_v[g, pl.ds(lbase, LANES)] & 1) * EMBED_DIM
                tot = jnp.zeros((LANES,), jnp.float32)
                for r in range(LANES):
                    row = lbase + r
                    uoff = upar[r]
                    moff = mpar[r]
                    acc = (urows[row, pl.ds(uoff, LANES)]
                           * mrows[row, pl.ds(moff, LANES)])
                    for c in range(1, DCHUNKS):
                        acc = acc + (
                            urows[row, pl.ds(uoff + c * LANES, LANES)]
                            * mrows[row, pl.ds(moff + c * LANES, LANES)])
                    for kk in (8, 4, 2, 1):
                        scr[pl.ds(0, LANES)] = acc
                        scr[pl.ds(LANES, LANES)] = acc
                        acc = acc + scr[pl.ds(kk, LANES)]
                    tot = jnp.where(lane_iota == r, acc, tot)
                out_v[pl.ds(g * CHUNK + lbase, LANES)] = tot
                return carry

            lax.fori_loop(0, GROUPS, group_body, 0)

        pending = {0: fire(0)}
        for g in range(NCHUNKS):
            if g + 1 < NCHUNKS:
                pending[g + 1] = fire(g + 1)
            for c in pending.pop(g):
                c.wait()
            compute_chunk(g)

        pltpu.sync_copy(out_v, out_hbm.at[pl.ds(wid * ROWS_PER_W, ROWS_PER_W)])

    return k(uidx2d, midx2d, user_pairs, movie_pairs)


def kernel(user_ids, movie_ids, side_info_ids, user_table, movie_table,
           side_table, alpha):
    del side_info_ids, side_table, alpha
    uidx = user_ids.astype(jnp.int32).reshape(BATCH // CHUNK, CHUNK)
    midx = movie_ids.astype(jnp.int32).reshape(BATCH // CHUNK, CHUNK)
    upairs = user_table.reshape(-1, PAIR)
    mpairs = movie_table.reshape(-1, PAIR)
    return _sc_similarity(uidx, midx, upairs, mpairs)


# speedup vs baseline: 2.1135x; 2.1135x over previous
import functools

import jax
import jax.numpy as jnp
from jax import lax
from jax.experimental import pallas as pl
from jax.experimental.pallas import tpu as pltpu
from jax.experimental.pallas import tpu_sc as plsc

BATCH = 16384
EMBED_DIM = 64
SLAB = 8
NC = 2
NS = 16
NW = NC * NS
ROWS_PER_W = BATCH // NW
CHUNK = 16
NCHUNKS = ROWS_PER_W // CHUNK
LANES = 16
DCHUNKS = EMBED_DIM // LANES
IDX_ROW = 128


def _sc_similarity(uidx2d, midx2d, utab3, mtab3):
    mesh = plsc.VectorSubcoreMesh(core_axis_name="c", subcore_axis_name="s")

    @functools.partial(
        pl.kernel,
        mesh=mesh,
        out_type=jax.ShapeDtypeStruct((BATCH,), jnp.float32),
        scratch_types=[
            pltpu.VMEM((ROWS_PER_W // IDX_ROW, IDX_ROW), jnp.int32),
            pltpu.VMEM((ROWS_PER_W // IDX_ROW, IDX_ROW), jnp.int32),
            pltpu.VMEM((NCHUNKS, CHUNK), jnp.int32),
            pltpu.VMEM((NCHUNKS, CHUNK), jnp.int32),
            pltpu.VMEM((2, CHUNK, SLAB, EMBED_DIM), jnp.float32),
            pltpu.VMEM((2, CHUNK, SLAB, EMBED_DIM), jnp.float32),
            pltpu.VMEM((2 * LANES,), jnp.float32),
            pltpu.VMEM((ROWS_PER_W,), jnp.float32),
            pltpu.SemaphoreType.DMA,
            pltpu.SemaphoreType.DMA,
        ],
    )
    def k(uidx_hbm, midx_hbm, utab_hbm, mtab_hbm, out_hbm,
          uidx_v, midx_v, uslab_v, mslab_v, ubuf, mbuf, scr, out_v,
          sem0, sem1):
        wid = lax.axis_index("s") * NC + lax.axis_index("c")
        ibase = wid * (ROWS_PER_W // IDX_ROW)

        pltpu.sync_copy(uidx_hbm.at[pl.ds(ibase, ROWS_PER_W // IDX_ROW)],
                        uidx_v)
        pltpu.sync_copy(midx_hbm.at[pl.ds(ibase, ROWS_PER_W // IDX_ROW)],
                        midx_v)

        def shr_body(j, carry):
            g = j // (IDX_ROW // LANES)
            o = (j % (IDX_ROW // LANES)) * LANES
            flat = j * LANES
            uslab_v[flat // CHUNK, pl.ds(0, LANES)] = jnp.right_shift(
                uidx_v[g, pl.ds(o, LANES)], 3)
            mslab_v[flat // CHUNK, pl.ds(0, LANES)] = jnp.right_shift(
                midx_v[g, pl.ds(o, LANES)], 3)
            return carry

        lax.fori_loop(0, ROWS_PER_W // LANES, shr_body, 0)

        sems = (sem0, sem1)

        def fire(g, slot):
            uvec = uslab_v[g, pl.ds(0, CHUNK)]
            mvec = mslab_v[g, pl.ds(0, CHUNK)]
            for r in range(CHUNK):
                pltpu.async_copy(utab_hbm.at[uvec[r]],
                                 ubuf.at[slot, r], sems[slot])
                pltpu.async_copy(mtab_hbm.at[mvec[r]],
                                 mbuf.at[slot, r], sems[slot])

        def drain(slot):
            pltpu.make_async_copy(utab_hbm.at[pl.ds(0, CHUNK)],
                                  ubuf.at[slot], sems[slot]).wait()
            pltpu.make_async_copy(mtab_hbm.at[pl.ds(0, CHUNK)],
                                  mbuf.at[slot], sems[slot]).wait()

        lane_iota = lax.iota(jnp.int32, LANES)

        def compute_chunk(g, slot):
            urows = ubuf.at[slot]
            mrows = mbuf.at[slot]
            uvec = uidx_v[g // (IDX_ROW // CHUNK),
                          pl.ds((g % (IDX_ROW // CHUNK)) * CHUNK, CHUNK)]
            mvec = midx_v[g // (IDX_ROW // CHUNK),
                          pl.ds((g % (IDX_ROW // CHUNK)) * CHUNK, CHUNK)]
            urr = uvec & (SLAB - 1)
            mrr = mvec & (SLAB - 1)
            tot = jnp.zeros((LANES,), jnp.float32)
            for r in range(CHUNK):
                ur = urr[r]
                mr = mrr[r]
                acc = (urows[r, ur, pl.ds(0, LANES)]
                       * mrows[r, mr, pl.ds(0, LANES)])
                for c in range(1, DCHUNKS):
                    acc = acc + (urows[r, ur, pl.ds(c * LANES, LANES)]
                                 * mrows[r, mr, pl.ds(c * LANES, LANES)])
                for kk in (8, 4, 2, 1):
                    scr[pl.ds(0, LANES)] = acc
                    scr[pl.ds(LANES, LANES)] = acc
                    acc = acc + scr[pl.ds(kk, LANES)]
                tot = jnp.where(lane_iota == r, acc, tot)
            out_v[pl.ds(g * CHUNK, CHUNK)] = tot

        fire(0, 0)

        def pair_body(i, carry):
            for sub in (0, 1):
                g = 2 * i + sub

                @pl.when(g + 1 < NCHUNKS)
                def _():
                    fire(g + 1, 1 - sub)

                drain(sub)
                compute_chunk(g, sub)
            return carry

        lax.fori_loop(0, NCHUNKS // 2, pair_body, 0)

        pltpu.sync_copy(out_v, out_hbm.at[pl.ds(wid * ROWS_PER_W, ROWS_PER_W)])

    return k(uidx2d, midx2d, utab3, mtab3)


def kernel(user_ids, movie_ids, side_info_ids, user_table, movie_table,
           side_table, alpha):
    del side_info_ids, side_table, alpha
    uidx = user_ids.astype(jnp.int32).reshape(BATCH // IDX_ROW, IDX_ROW)
    midx = movie_ids.astype(jnp.int32).reshape(BATCH // IDX_ROW, IDX_ROW)
    utab3 = user_table.reshape(-1, SLAB, EMBED_DIM)
    mtab3 = movie_table.reshape(-1, SLAB, EMBED_DIM)
    return _sc_similarity(uidx, midx, utab3, mtab3)


# speedup vs baseline: 2.1540x; 1.0191x over previous
import functools

import jax
import jax.numpy as jnp
from jax import lax
from jax.experimental import pallas as pl
from jax.experimental.pallas import tpu as pltpu
from jax.experimental.pallas import tpu_sc as plsc

BATCH = 16384
EMBED_DIM = 64
SLAB = 8
NC = 2
NS = 16
NW = NC * NS
ROWS_PER_W = BATCH // NW
CHUNK = 16
NCHUNKS = ROWS_PER_W // CHUNK
LANES = 16
DCHUNKS = EMBED_DIM // LANES
IDX_ROW = 128


def _sc_similarity(uidx2d, midx2d, utab3, mtab3):
    mesh = plsc.VectorSubcoreMesh(core_axis_name="c", subcore_axis_name="s")

    @functools.partial(
        pl.kernel,
        mesh=mesh,
        out_type=jax.ShapeDtypeStruct((BATCH,), jnp.float32),
        scratch_types=[
            pltpu.VMEM((ROWS_PER_W // IDX_ROW, IDX_ROW), jnp.int32),
            pltpu.VMEM((ROWS_PER_W // IDX_ROW, IDX_ROW), jnp.int32),
            pltpu.VMEM((NCHUNKS, CHUNK), jnp.int32),
            pltpu.VMEM((NCHUNKS, CHUNK), jnp.int32),
            pltpu.VMEM((2, CHUNK, SLAB, EMBED_DIM), jnp.float32),
            pltpu.VMEM((2, CHUNK, 2 * EMBED_DIM), jnp.float32),
            pltpu.VMEM((2 * LANES,), jnp.float32),
            pltpu.VMEM((ROWS_PER_W,), jnp.float32),
            pltpu.SemaphoreType.DMA,
            pltpu.SemaphoreType.DMA,
        ],
    )
    def k(uidx_hbm, midx_hbm, utab_hbm, mtab_hbm, out_hbm,
          uidx_v, midx_v, uslab_v, mslab_v, ubuf, mbuf, scr, out_v,
          sem0, sem1):
        wid = lax.axis_index("s") * NC + lax.axis_index("c")
        ibase = wid * (ROWS_PER_W // IDX_ROW)

        pltpu.sync_copy(uidx_hbm.at[pl.ds(ibase, ROWS_PER_W // IDX_ROW)],
                        uidx_v)
        pltpu.sync_copy(midx_hbm.at[pl.ds(ibase, ROWS_PER_W // IDX_ROW)],
                        midx_v)

        def shr_body(j, carry):
            g = j // (IDX_ROW // LANES)
            o = (j % (IDX_ROW // LANES)) * LANES
            flat = j * LANES
            uslab_v[flat // CHUNK, pl.ds(0, LANES)] = jnp.right_shift(
                uidx_v[g, pl.ds(o, LANES)], 3)
            mslab_v[flat // CHUNK, pl.ds(0, LANES)] = jnp.right_shift(
                midx_v[g, pl.ds(o, LANES)], 1)
            return carry

        lax.fori_loop(0, ROWS_PER_W // LANES, shr_body, 0)

        sems = (sem0, sem1)

        def fire(g, slot):
            uvec = uslab_v[g, pl.ds(0, CHUNK)]
            for r in range(CHUNK):
                pltpu.async_copy(utab_hbm.at[uvec[r]],
                                 ubuf.at[slot, r], sems[slot])
            pltpu.async_copy(mtab_hbm.at[mslab_v.at[g]],
                             mbuf.at[slot], sems[slot])

        def drain(slot):
            pltpu.make_async_copy(utab_hbm.at[pl.ds(0, CHUNK)],
                                  ubuf.at[slot], sems[slot]).wait()
            pltpu.make_async_copy(mtab_hbm.at[pl.ds(0, CHUNK)],
                                  mbuf.at[slot], sems[slot]).wait()

        lane_iota = lax.iota(jnp.int32, LANES)

        def compute_chunk(g, slot):
            urows = ubuf.at[slot]
            mrows = mbuf.at[slot]
            uvec = uidx_v[g // (IDX_ROW // CHUNK),
                          pl.ds((g % (IDX_ROW // CHUNK)) * CHUNK, CHUNK)]
            mvec = midx_v[g // (IDX_ROW // CHUNK),
                          pl.ds((g % (IDX_ROW // CHUNK)) * CHUNK, CHUNK)]
            urr = uvec & (SLAB - 1)
            mhh = (mvec & 1) * EMBED_DIM
            tot = jnp.zeros((LANES,), jnp.float32)
            for r in range(CHUNK):
                ur = urr[r]
                mo = mhh[r]
                acc = (urows[r, ur, pl.ds(0, LANES)]
                       * mrows[r, pl.ds(mo, LANES)])
                for c in range(1, DCHUNKS):
                    acc = acc + (urows[r, ur, pl.ds(c * LANES, LANES)]
                                 * mrows[r, pl.ds(mo + c * LANES, LANES)])
                for kk in (8, 4, 2, 1):
                    scr[pl.ds(0, LANES)] = acc
                    scr[pl.ds(LANES, LANES)] = acc
                    acc = acc + scr[pl.ds(kk, LANES)]
                tot = jnp.where(lane_iota == r, acc, tot)
            out_v[pl.ds(g * CHUNK, CHUNK)] = tot

        fire(0, 0)

        def pair_body(i, carry):
            for sub in (0, 1):
                g = 2 * i + sub

                @pl.when(g + 1 < NCHUNKS)
                def _():
                    fire(g + 1, 1 - sub)

                drain(sub)
                compute_chunk(g, sub)
            return carry

        lax.fori_loop(0, NCHUNKS // 2, pair_body, 0)

        pltpu.sync_copy(out_v, out_hbm.at[pl.ds(wid * ROWS_PER_W, ROWS_PER_W)])

    return k(uidx2d, midx2d, utab3, mtab3)


def kernel(user_ids, movie_ids, side_info_ids, user_table, movie_table,
           side_table, alpha):
    del side_info_ids, side_table, alpha
    uidx = user_ids.astype(jnp.int32).reshape(BATCH // IDX_ROW, IDX_ROW)
    midx = movie_ids.astype(jnp.int32).reshape(BATCH // IDX_ROW, IDX_ROW)
    utab3 = user_table.reshape(-1, SLAB, EMBED_DIM)
    mtab2 = movie_table.reshape(-1, 2 * EMBED_DIM)
    return _sc_similarity(uidx, midx, utab3, mtab2)


# speedup vs baseline: 2.1567x; 1.0012x over previous
import functools

import jax
import jax.numpy as jnp
from jax import lax
from jax.experimental import pallas as pl
from jax.experimental.pallas import tpu as pltpu
from jax.experimental.pallas import tpu_sc as plsc

BATCH = 16384
EMBED_DIM = 64
SLAB = 8
NC = 2
NS = 16
NW = NC * NS
ROWS_PER_W = BATCH // NW
CHUNK = 16
NCHUNKS = ROWS_PER_W // CHUNK
LANES = 16
DCHUNKS = EMBED_DIM // LANES
IDX_ROW = 128


def _sc_similarity(uidx2d, midx2d, utab3, mtab3):
    mesh = plsc.VectorSubcoreMesh(core_axis_name="c", subcore_axis_name="s")

    @functools.partial(
        pl.kernel,
        mesh=mesh,
        out_type=jax.ShapeDtypeStruct((BATCH,), jnp.float32),
        scratch_types=[
            pltpu.VMEM((ROWS_PER_W // IDX_ROW, IDX_ROW), jnp.int32),
            pltpu.VMEM((ROWS_PER_W // IDX_ROW, IDX_ROW), jnp.int32),
            pltpu.VMEM((NCHUNKS, CHUNK), jnp.int32),
            pltpu.VMEM((NCHUNKS, CHUNK), jnp.int32),
            pltpu.VMEM((4, CHUNK, SLAB, EMBED_DIM), jnp.float32),
            pltpu.VMEM((4, CHUNK, 2 * EMBED_DIM), jnp.float32),
            pltpu.VMEM((2 * LANES,), jnp.float32),
            pltpu.VMEM((ROWS_PER_W,), jnp.float32),
            pltpu.SemaphoreType.DMA,
            pltpu.SemaphoreType.DMA,
            pltpu.SemaphoreType.DMA,
            pltpu.SemaphoreType.DMA,
        ],
    )
    def k(uidx_hbm, midx_hbm, utab_hbm, mtab_hbm, out_hbm,
          uidx_v, midx_v, uslab_v, mslab_v, ubuf, mbuf, scr, out_v,
          sem0, sem1, sem2, sem3):
        wid = lax.axis_index("s") * NC + lax.axis_index("c")
        ibase = wid * (ROWS_PER_W // IDX_ROW)

        pltpu.sync_copy(uidx_hbm.at[pl.ds(ibase, ROWS_PER_W // IDX_ROW)],
                        uidx_v)
        pltpu.sync_copy(midx_hbm.at[pl.ds(ibase, ROWS_PER_W // IDX_ROW)],
                        midx_v)

        def shr_body(j, carry):
            g = j // (IDX_ROW // LANES)
            o = (j % (IDX_ROW // LANES)) * LANES
            flat = j * LANES
            uslab_v[flat // CHUNK, pl.ds(0, LANES)] = jnp.right_shift(
                uidx_v[g, pl.ds(o, LANES)], 3)
            mslab_v[flat // CHUNK, pl.ds(0, LANES)] = jnp.right_shift(
                midx_v[g, pl.ds(o, LANES)], 1)
            return carry

        lax.fori_loop(0, ROWS_PER_W // LANES, shr_body, 0)

        sems = (sem0, sem1, sem2, sem3)

        def fire(g, slot):
            uvec = uslab_v[g, pl.ds(0, CHUNK)]
            for r in range(CHUNK):
                pltpu.async_copy(utab_hbm.at[uvec[r]],
                                 ubuf.at[slot, r], sems[slot])
            pltpu.async_copy(mtab_hbm.at[mslab_v.at[g]],
                             mbuf.at[slot], sems[slot])

        def drain(slot):
            pltpu.make_async_copy(utab_hbm.at[pl.ds(0, CHUNK)],
                                  ubuf.at[slot], sems[slot]).wait()
            pltpu.make_async_copy(mtab_hbm.at[pl.ds(0, CHUNK)],
                                  mbuf.at[slot], sems[slot]).wait()

        lane_iota = lax.iota(jnp.int32, LANES)

        def compute_chunk(g, slot):
            urows = ubuf.at[slot]
            mrows = mbuf.at[slot]
            uvec = uidx_v[g // (IDX_ROW // CHUNK),
                          pl.ds((g % (IDX_ROW // CHUNK)) * CHUNK, CHUNK)]
            mvec = midx_v[g // (IDX_ROW // CHUNK),
                          pl.ds((g % (IDX_ROW // CHUNK)) * CHUNK, CHUNK)]
            urr = uvec & (SLAB - 1)
            mhh = (mvec & 1) * EMBED_DIM
            tot = jnp.zeros((LANES,), jnp.float32)
            for r in range(CHUNK):
                ur = urr[r]
                mo = mhh[r]
                acc = (urows[r, ur, pl.ds(0, LANES)]
                       * mrows[r, pl.ds(mo, LANES)])
                for c in range(1, DCHUNKS):
                    acc = acc + (urows[r, ur, pl.ds(c * LANES, LANES)]
                                 * mrows[r, pl.ds(mo + c * LANES, LANES)])
                for kk in (8, 4, 2, 1):
                    scr[pl.ds(0, LANES)] = acc
                    scr[pl.ds(LANES, LANES)] = acc
                    acc = acc + scr[pl.ds(kk, LANES)]
                tot = jnp.where(lane_iota == r, acc, tot)
            out_v[pl.ds(g * CHUNK, CHUNK)] = tot

        fire(0, 0)
        fire(1, 1)

        def quad_body(i, carry):
            for sub in range(4):
                g = 4 * i + sub

                @pl.when(g + 2 < NCHUNKS)
                def _():
                    fire(g + 2, (sub + 2) % 4)

                drain(sub)
                compute_chunk(g, sub)
            return carry

        lax.fori_loop(0, NCHUNKS // 4, quad_body, 0)

        pltpu.sync_copy(out_v, out_hbm.at[pl.ds(wid * ROWS_PER_W, ROWS_PER_W)])

    return k(uidx2d, midx2d, utab3, mtab3)


def kernel(user_ids, movie_ids, side_info_ids, user_table, movie_table,
           side_table, alpha):
    del side_info_ids, side_table, alpha
    uidx = user_ids.astype(jnp.int32).reshape(BATCH // IDX_ROW, IDX_ROW)
    midx = movie_ids.astype(jnp.int32).reshape(BATCH // IDX_ROW, IDX_ROW)
    utab3 = user_table.reshape(-1, SLAB, EMBED_DIM)
    mtab2 = movie_table.reshape(-1, 2 * EMBED_DIM)
    return _sc_similarity(uidx, midx, utab3, mtab2)
